# initial kernel scaffold (unmeasured)
import jax
import jax.numpy as jnp
from jax import lax
from jax.experimental import pallas as pl
from jax.experimental.pallas import tpu as pltpu

N_DEV = 16
M_CH = 256
N_HALF = 1024
N_SLOTS = 3
N_HOPS = 2 * (N_DEV - 1)


def kernel(x, w_mat):
    m, _ = x.shape
    _, n = w_mat.shape

    def body(x_ref, w_ref, out_ref, wbf_ref, comm_f, comm_r,
             send_f, recv_f, send_r, recv_r):
        my = lax.axis_index("i")
        right = jnp.mod(my + 1, N_DEV)
        left = jnp.mod(my - 1, N_DEV)

        wbf_ref[...] = w_ref[...].astype(jnp.bfloat16)

        def pchunk(c, col0):
            xa = x_ref[pl.ds(c * M_CH, M_CH), :].astype(jnp.bfloat16)
            return jnp.dot(xa, wbf_ref[:, col0:col0 + N_HALF],
                           preferred_element_type=jnp.float32)

        comm_f[0] = pchunk(my, 0).astype(jnp.bfloat16)
        comm_r[0] = pchunk(my, N_HALF).astype(jnp.bfloat16)

        bar = pltpu.get_barrier_semaphore()
        for nbr in (left, right):
            pltpu.semaphore_signal(bar, inc=1, device_id=(nbr,),
                                   device_id_type=pl.DeviceIdType.MESH)
        pltpu.semaphore_wait(bar, 2)

        prev = None
        for h in range(N_HOPS):
            ss = h % N_SLOTS
            rs = (h + 1) % N_SLOTS
            if prev is not None:
                prev[0].wait_send()
                prev[1].wait_send()
            rdma_f = pltpu.make_async_remote_copy(
                src_ref=comm_f.at[ss], dst_ref=comm_f.at[rs],
                send_sem=send_f.at[ss], recv_sem=recv_f.at[rs],
                device_id=(right,), device_id_type=pl.DeviceIdType.MESH)
            rdma_r = pltpu.make_async_remote_copy(
                src_ref=comm_r.at[ss], dst_ref=comm_r.at[rs],
                send_sem=send_r.at[ss], recv_sem=recv_r.at[rs],
                device_id=(left,), device_id_type=pl.DeviceIdType.MESH)
            rdma_f.start()
            rdma_r.start()
            prev = (rdma_f, rdma_r)
            rdma_f.wait_recv()
            rdma_r.wait_recv()

            if h < N_DEV - 2:
                rf = jnp.mod(my - h - 1, N_DEV)
                rr = jnp.mod(my + h + 1, N_DEV)
                comm_f[rs] = (comm_f[rs].astype(jnp.float32)
                              + pchunk(rf, 0)).astype(jnp.bfloat16)
                comm_r[rs] = (comm_r[rs].astype(jnp.float32)
                              + pchunk(rr, N_HALF)).astype(jnp.bfloat16)
            elif h == N_DEV - 2:
                rf = jnp.mod(my + 1, N_DEV)
                rr = jnp.mod(my - 1, N_DEV)
                vf = jnp.maximum(comm_f[rs].astype(jnp.float32)
                                 + pchunk(rf, 0), 0.0)
                vr = jnp.maximum(comm_r[rs].astype(jnp.float32)
                                 + pchunk(rr, N_HALF), 0.0)
                comm_f[rs] = vf.astype(jnp.bfloat16)
                comm_r[rs] = vr.astype(jnp.bfloat16)
                out_ref[pl.ds(rf * M_CH, M_CH), 0:N_HALF] = (
                    comm_f[rs].astype(jnp.float32))
                out_ref[pl.ds(rr * M_CH, M_CH), N_HALF:n] = (
                    comm_r[rs].astype(jnp.float32))
            else:
                t = h - (N_DEV - 1)
                cf = jnp.mod(my - t, N_DEV)
                cr = jnp.mod(my + t, N_DEV)
                out_ref[pl.ds(cf * M_CH, M_CH), 0:N_HALF] = (
                    comm_f[rs].astype(jnp.float32))
                out_ref[pl.ds(cr * M_CH, M_CH), N_HALF:n] = (
                    comm_r[rs].astype(jnp.float32))

        prev[0].wait_send()
        prev[1].wait_send()

    return pl.pallas_call(
        body,
        out_shape=jax.ShapeDtypeStruct((m, n), jnp.float32),
        in_specs=[pl.BlockSpec(memory_space=pltpu.VMEM),
                  pl.BlockSpec(memory_space=pltpu.VMEM)],
        out_specs=pl.BlockSpec(memory_space=pltpu.VMEM),
        scratch_shapes=[
            pltpu.VMEM((k_shard_k(x), n), jnp.bfloat16),
            pltpu.VMEM((N_SLOTS, M_CH, N_HALF), jnp.bfloat16),
            pltpu.VMEM((N_SLOTS, M_CH, N_HALF), jnp.bfloat16),
            pltpu.SemaphoreType.DMA((N_SLOTS,)),
            pltpu.SemaphoreType.DMA((N_SLOTS,)),
            pltpu.SemaphoreType.DMA((N_SLOTS,)),
            pltpu.SemaphoreType.DMA((N_SLOTS,)),
        ],
        compiler_params=pltpu.CompilerParams(collective_id=0),
    )(x, w_mat)


def k_shard_k(x):
    return x.shape[1]


# baseline (device time: 314239 ns/iter reference)
import jax
import jax.numpy as jnp
from jax import lax
from jax.experimental import pallas as pl
from jax.experimental.pallas import tpu as pltpu

N_DEV = 16
M_CH = 256
N_HALF = 1024
N_SLOTS = 3
N_HOPS = 2 * (N_DEV - 1)


def kernel(x, w_mat):
    m, _ = x.shape
    k_sh, n = w_mat.shape

    def body(x_ref, w_ref, out_ref, wbf_ref, comm_f, comm_r,
             send_f, recv_f, send_r, recv_r):
        my = lax.axis_index("i")
        right = jnp.mod(my + 1, N_DEV)
        left = jnp.mod(my - 1, N_DEV)

        wbf_ref[...] = w_ref[...].astype(jnp.bfloat16)

        def pchunk(c, col0):
            xa = x_ref[pl.ds(c * M_CH, M_CH), :].astype(jnp.bfloat16)
            return jnp.dot(xa, wbf_ref[:, col0:col0 + N_HALF],
                           preferred_element_type=jnp.float32)

        comm_f[0] = pchunk(my, 0).astype(jnp.bfloat16)
        comm_r[0] = pchunk(my, N_HALF).astype(jnp.bfloat16)

        bar = pltpu.get_barrier_semaphore()
        for nbr in (left, right):
            pltpu.semaphore_signal(bar, inc=1, device_id=(nbr,),
                                   device_id_type=pl.DeviceIdType.MESH)
        pltpu.semaphore_wait(bar, 2)

        prev = None
        for h in range(N_HOPS):
            ss = h % N_SLOTS
            rs = (h + 1) % N_SLOTS
            if prev is not None:
                prev[0].wait_send()
                prev[1].wait_send()
            rdma_f = pltpu.make_async_remote_copy(
                src_ref=comm_f.at[ss], dst_ref=comm_f.at[rs],
                send_sem=send_f.at[ss], recv_sem=recv_f.at[rs],
                device_id=(right,), device_id_type=pl.DeviceIdType.MESH)
            rdma_r = pltpu.make_async_remote_copy(
                src_ref=comm_r.at[ss], dst_ref=comm_r.at[rs],
                send_sem=send_r.at[ss], recv_sem=recv_r.at[rs],
                device_id=(left,), device_id_type=pl.DeviceIdType.MESH)
            rdma_f.start()
            rdma_r.start()
            prev = (rdma_f, rdma_r)
            rdma_f.wait_recv()
            rdma_r.wait_recv()

            if h < N_DEV - 2:
                rf = jnp.mod(my - h - 1, N_DEV)
                rr = jnp.mod(my + h + 1, N_DEV)
                comm_f[rs] = (comm_f[rs].astype(jnp.float32)
                              + pchunk(rf, 0)).astype(jnp.bfloat16)
                comm_r[rs] = (comm_r[rs].astype(jnp.float32)
                              + pchunk(rr, N_HALF)).astype(jnp.bfloat16)
            elif h == N_DEV - 2:
                rf = jnp.mod(my + 1, N_DEV)
                rr = jnp.mod(my - 1, N_DEV)
                vf = jnp.maximum(comm_f[rs].astype(jnp.float32)
                                 + pchunk(rf, 0), 0.0)
                vr = jnp.maximum(comm_r[rs].astype(jnp.float32)
                                 + pchunk(rr, N_HALF), 0.0)
                comm_f[rs] = vf.astype(jnp.bfloat16)
                comm_r[rs] = vr.astype(jnp.bfloat16)
                out_ref[pl.ds(rf * M_CH, M_CH), 0:N_HALF] = (
                    comm_f[rs].astype(jnp.float32))
                out_ref[pl.ds(rr * M_CH, M_CH), N_HALF:n] = (
                    comm_r[rs].astype(jnp.float32))
            else:
                t = h - (N_DEV - 1)
                cf = jnp.mod(my - t, N_DEV)
                cr = jnp.mod(my + t, N_DEV)
                out_ref[pl.ds(cf * M_CH, M_CH), 0:N_HALF] = (
                    comm_f[rs].astype(jnp.float32))
                out_ref[pl.ds(cr * M_CH, M_CH), N_HALF:n] = (
                    comm_r[rs].astype(jnp.float32))

        prev[0].wait_send()
        prev[1].wait_send()

    return pl.pallas_call(
        body,
        out_shape=jax.ShapeDtypeStruct((m, n), jnp.float32),
        in_specs=[pl.BlockSpec(memory_space=pltpu.VMEM),
                  pl.BlockSpec(memory_space=pltpu.VMEM)],
        out_specs=pl.BlockSpec(memory_space=pltpu.VMEM),
        scratch_shapes=[
            pltpu.VMEM((k_sh, n), jnp.bfloat16),
            pltpu.VMEM((N_SLOTS, M_CH, N_HALF), jnp.bfloat16),
            pltpu.VMEM((N_SLOTS, M_CH, N_HALF), jnp.bfloat16),
            pltpu.SemaphoreType.DMA((N_SLOTS,)),
            pltpu.SemaphoreType.DMA((N_SLOTS,)),
            pltpu.SemaphoreType.DMA((N_SLOTS,)),
            pltpu.SemaphoreType.DMA((N_SLOTS,)),
        ],
        compiler_params=pltpu.CompilerParams(
            collective_id=0, vmem_limit_bytes=56 * 1024 * 1024),
    )(x, w_mat)


# device time: 214591 ns/iter; 1.4644x vs baseline; 1.4644x over previous
import jax
import jax.numpy as jnp
from jax import lax
from jax.experimental import pallas as pl
from jax.experimental.pallas import tpu as pltpu

N_DEV = 16
M_CH = 256
N_HALF = 1024
SUB = 4
SUB_M = M_CH // SUB
N_RINGS = 2 * SUB
N_SLOTS = 5
N_HOPS = 2 * (N_DEV - 1)


def kernel(x, w_mat):
    m, _ = x.shape
    k_sh, n = w_mat.shape

    def body(x_ref, w_ref, out_ref, wbf_ref, comm, send_sems, recv_sems):
        my = lax.axis_index("i")
        right = jnp.mod(my + 1, N_DEV)
        left = jnp.mod(my - 1, N_DEV)

        wbf_ref[...] = w_ref[...].astype(jnp.bfloat16)

        def pchunk(c, col0):
            xa = x_ref[pl.ds(c * M_CH, M_CH), :].astype(jnp.bfloat16)
            return jnp.dot(xa, wbf_ref[:, col0:col0 + N_HALF],
                           preferred_element_type=jnp.float32)

        def ring_dir(t):
            return 1 if t % 2 == 0 else -1

        def make_rdma(t, h):
            tgt = right if ring_dir(t) == 1 else left
            ss, rs = h % N_SLOTS, (h + 1) % N_SLOTS
            return pltpu.make_async_remote_copy(
                src_ref=comm.at[t, ss], dst_ref=comm.at[t, rs],
                send_sem=send_sems.at[t, ss], recv_sem=recv_sems.at[t, rs],
                device_id=(tgt,), device_id_type=pl.DeviceIdType.MESH)

        p_f = pchunk(my, 0)
        p_r = pchunk(my, N_HALF)
        for t in range(N_RINGS):
            s = t // 2
            p = p_f if t % 2 == 0 else p_r
            comm[t, 0] = p[s * SUB_M:(s + 1) * SUB_M, :].astype(jnp.bfloat16)

        bar = pltpu.get_barrier_semaphore()
        for nbr in (left, right):
            pltpu.semaphore_signal(bar, inc=1, device_id=(nbr,),
                                   device_id_type=pl.DeviceIdType.MESH)
        pltpu.semaphore_wait(bar, 2)

        rdmas = {}
        for t in range(N_RINGS):
            rdmas[(0, t)] = make_rdma(t, 0)
            rdmas[(0, t)].start()

        def process(t, h, rf, rr, p_f, p_r):
            sig, s = ring_dir(t), t // 2
            rs = (h + 1) % N_SLOTS
            r = rf if sig == 1 else rr
            col0 = 0 if sig == 1 else N_HALF
            row = r * M_CH + s * SUB_M
            if h < N_DEV - 2:
                p = (p_f if sig == 1 else p_r)[s * SUB_M:(s + 1) * SUB_M, :]
                comm[t, rs] = (comm[t, rs].astype(jnp.float32)
                               + p).astype(jnp.bfloat16)
            elif h == N_DEV - 2:
                p = (p_f if sig == 1 else p_r)[s * SUB_M:(s + 1) * SUB_M, :]
                v = jnp.maximum(comm[t, rs].astype(jnp.float32) + p, 0.0)
                comm[t, rs] = v.astype(jnp.bfloat16)
                out_ref[pl.ds(row, SUB_M), col0:col0 + N_HALF] = (
                    comm[t, rs].astype(jnp.float32))
            else:
                out_ref[pl.ds(row, SUB_M), col0:col0 + N_HALF] = (
                    comm[t, rs].astype(jnp.float32))

        for h in range(1, N_HOPS + 1):
            step = h - 1
            if step < N_DEV - 1:
                rf = jnp.mod(my - step - 1, N_DEV)
                rr = jnp.mod(my + step + 1, N_DEV)
                p_f = pchunk(rf, 0)
                p_r = pchunk(rr, N_HALF)
            else:
                tt = step - (N_DEV - 1)
                rf = jnp.mod(my - tt, N_DEV)
                rr = jnp.mod(my + tt, N_DEV)
                p_f = p_r = None
            for t in range(N_RINGS):
                rdmas[(h - 1, t)].wait_recv()
                process(t, h - 1, rf, rr, p_f, p_r)
                if h <= N_HOPS - 1:
                    if h >= 2:
                        rdmas[(h - 2, t)].wait_send()
                    rdmas[(h, t)] = make_rdma(t, h)
                    rdmas[(h, t)].start()
        for t in range(N_RINGS):
            rdmas[(N_HOPS - 2, t)].wait_send()
            rdmas[(N_HOPS - 1, t)].wait_send()

    return pl.pallas_call(
        body,
        out_shape=jax.ShapeDtypeStruct((m, n), jnp.float32),
        in_specs=[pl.BlockSpec(memory_space=pltpu.VMEM),
                  pl.BlockSpec(memory_space=pltpu.VMEM)],
        out_specs=pl.BlockSpec(memory_space=pltpu.VMEM),
        scratch_shapes=[
            pltpu.VMEM((k_sh, n), jnp.bfloat16),
            pltpu.VMEM((N_RINGS, N_SLOTS, SUB_M, N_HALF), jnp.bfloat16),
            pltpu.SemaphoreType.DMA((N_RINGS, N_SLOTS)),
            pltpu.SemaphoreType.DMA((N_RINGS, N_SLOTS)),
        ],
        compiler_params=pltpu.CompilerParams(
            collective_id=0, vmem_limit_bytes=56 * 1024 * 1024),
    )(x, w_mat)
